# baseline (speedup 1.0000x reference)
import jax
import jax.numpy as jnp
from jax import lax
from jax.experimental import pallas as pl
from jax.experimental.pallas import tpu as pltpu
from jax.experimental.pallas import tpu_sc as plsc

BATCH = 4096
SEQ = 50
DIM = 64
PADDING_IDX = 0

NUM_CORES = 2
NUM_SUBCORES = 16
NUM_WORKERS = NUM_CORES * NUM_SUBCORES

BPW = BATCH // NUM_WORKERS
LANES = 16
COLV = DIM // LANES
GROUPS = BPW // LANES

NBUF = 5
ROUNDS = SEQ // NBUF


def _lookup_body(table_hbm, idx_hbm, out_hbm, idx_v, g_v, t_v, *sems):
    gsems, ssems = sems[:NBUF], sems[NBUF:]
    wid = lax.axis_index("s") * NUM_CORES + lax.axis_index("c")
    c0 = wid * BPW
    pltpu.sync_copy(idx_hbm.at[:, pl.ds(c0, BPW)], idx_v)

    def gather_desc(l, slot):
        return pltpu.make_async_copy(
            table_hbm.at[idx_v.at[l]], g_v.at[slot], gsems[slot])

    def store_desc(l, slot):
        return pltpu.make_async_copy(
            t_v.at[slot], out_hbm.at[l].at[:, wid], ssems[slot])

    def fixup(l, slot):
        vmin = idx_v[l, pl.ds(0, LANES)]
        for g in range(1, GROUPS):
            vmin = jnp.minimum(vmin, idx_v[l, pl.ds(g * LANES, LANES)])
        smin = vmin[0]
        for i in range(1, LANES):
            smin = jnp.minimum(smin, vmin[i])

        @pl.when(smin == PADDING_IDX)
        def _fix():
            def grp_body(g, c2):
                idx16 = idx_v[l, pl.ds(g * LANES, LANES)]
                for i in range(LANES):

                    @pl.when(idx16[i] == PADDING_IDX)
                    def _zero(i=i):
                        r = g * LANES + i
                        for c in range(COLV):
                            g_v[slot, r, pl.ds(c * LANES, LANES)] = jnp.zeros(
                                (LANES,), jnp.float32)

                return c2

            lax.fori_loop(0, GROUPS, grp_body, 0)

    iota = jax.lax.iota(jnp.int32, LANES)
    t_ks = [(iota + k) & (LANES - 1) for k in range(LANES)]

    def transpose(slot):
        def g_body(g, c1):
            rowv = g * LANES + iota
            for k in range(LANES):
                t = t_ks[k]
                for c in range(COLV):
                    d = jnp.bitwise_or(t, c * LANES)
                    v = plsc.load_gather(g_v.at[slot], [rowv, d])
                    plsc.store_scatter(
                        t_v.at[slot],
                        [jax.lax.shift_right_logical(d, 3),
                         jnp.bitwise_and(d, 7), rowv], v)
            return c1

        lax.fori_loop(0, GROUPS, g_body, 0)

    for b in range(NBUF):
        gather_desc(b, b).start()

    def round_body(t, carry):
        for b in range(NBUF):
            l = t * NBUF + b
            gather_desc(l, b).wait()
            fixup(l, b)

            @pl.when(t > 0)
            def _drain_prev():
                store_desc(l - NBUF, b).wait()

            transpose(b)
            store_desc(l, b).start()

            @pl.when(t < ROUNDS - 1)
            def _issue_next():
                gather_desc(l + NBUF, b).start()

        return carry

    lax.fori_loop(0, ROUNDS, round_body, 0)

    for b in range(NBUF):
        store_desc((ROUNDS - 1) * NBUF + b, b).wait()


_lookup = pl.kernel(
    _lookup_body,
    out_type=jax.ShapeDtypeStruct(
        (SEQ, DIM // 8, NUM_WORKERS, 8, BPW), jnp.float32),
    mesh=plsc.VectorSubcoreMesh(core_axis_name="c", subcore_axis_name="s"),
    compiler_params=pltpu.CompilerParams(
        use_tc_tiling_on_sc=False, needs_layout_passes=False),
    scratch_types=[
        pltpu.VMEM((SEQ, BPW), jnp.int32),
        pltpu.VMEM((NBUF, BPW, DIM), jnp.float32),
        pltpu.VMEM((NBUF, DIM // 8, 8, BPW), jnp.float32),
    ] + [pltpu.SemaphoreType.DMA] * (2 * NBUF),
)


def kernel(input_batch, table):
    out5 = _lookup(table, input_batch.T.astype(jnp.int32))
    return jnp.transpose(out5, (2, 4, 0, 1, 3)).reshape(BATCH, SEQ, DIM)


# speedup vs baseline: 1.4599x; 1.4599x over previous
import jax
import jax.numpy as jnp
from jax import lax
from jax.experimental import pallas as pl
from jax.experimental.pallas import tpu as pltpu
from jax.experimental.pallas import tpu_sc as plsc

BATCH = 4096
SEQ = 50
DIM = 64
PADDING_IDX = 0

NUM_CORES = 2
NUM_SUBCORES = 16
NUM_WORKERS = NUM_CORES * NUM_SUBCORES

BPW = BATCH // NUM_WORKERS
LANES = 16
COLV = DIM // LANES
GROUPS = BPW // LANES

NBUF = 5
ROUNDS = SEQ // NBUF


def _lookup_body(table_hbm, idx_hbm, out_hbm, idx_v, g_v, t_v, *sems):
    gsems, ssems = sems[:NBUF], sems[NBUF:]
    wid = lax.axis_index("s") * NUM_CORES + lax.axis_index("c")
    c0 = wid * BPW
    pltpu.sync_copy(idx_hbm.at[:, pl.ds(c0, BPW)], idx_v)

    def gather_desc(l, slot):
        return pltpu.make_async_copy(
            table_hbm.at[idx_v.at[l]], g_v.at[slot], gsems[slot])

    def store_desc(l, slot):
        return pltpu.make_async_copy(
            t_v.at[slot], out_hbm.at[l].at[:, wid], ssems[slot])

    def fixup(l, slot):
        vmin = idx_v[l, pl.ds(0, LANES)]
        for g in range(1, GROUPS):
            vmin = jnp.minimum(vmin, idx_v[l, pl.ds(g * LANES, LANES)])
        smin = vmin[0]
        for i in range(1, LANES):
            smin = jnp.minimum(smin, vmin[i])

        @pl.when(smin == PADDING_IDX)
        def _fix():
            def grp_body(g, c2):
                idx16 = idx_v[l, pl.ds(g * LANES, LANES)]
                for i in range(LANES):

                    @pl.when(idx16[i] == PADDING_IDX)
                    def _zero(i=i):
                        r = g * LANES + i
                        for c in range(COLV):
                            g_v[slot, r, pl.ds(c * LANES, LANES)] = jnp.zeros(
                                (LANES,), jnp.float32)

                return c2

            lax.fori_loop(0, GROUPS, grp_body, 0)

    iota = jax.lax.iota(jnp.int32, LANES)
    t_ks = [(iota + k) & (LANES - 1) for k in range(LANES)]

    def transpose(slot):
        def g_body(g, c1):
            rowv = g * LANES + iota
            for k in range(0, LANES, 2):
                ds_ = [jnp.bitwise_or(t_ks[k + j], c * LANES)
                       for j in range(2) for c in range(COLV)]
                vals = [plsc.load_gather(g_v.at[slot], [rowv, d])
                        for d in ds_]
                for d, v in zip(ds_, vals):
                    plsc.store_scatter(
                        t_v.at[slot],
                        [jax.lax.shift_right_logical(d, 3),
                         jnp.bitwise_and(d, 7), rowv], v)
            return c1

        lax.fori_loop(0, GROUPS, g_body, 0)

    for b in range(NBUF):
        gather_desc(b, b).start()

    def round_body(t, carry):
        for b in range(NBUF):
            l = t * NBUF + b
            gather_desc(l, b).wait()
            fixup(l, b)

            @pl.when(t > 0)
            def _drain_prev():
                store_desc(l - NBUF, b).wait()

            transpose(b)
            store_desc(l, b).start()

            @pl.when(t < ROUNDS - 1)
            def _issue_next():
                gather_desc(l + NBUF, b).start()

        return carry

    lax.fori_loop(0, ROUNDS, round_body, 0)

    for b in range(NBUF):
        store_desc((ROUNDS - 1) * NBUF + b, b).wait()


_lookup = pl.kernel(
    _lookup_body,
    out_type=jax.ShapeDtypeStruct(
        (SEQ, DIM // 8, NUM_WORKERS, 8, BPW), jnp.float32),
    mesh=plsc.VectorSubcoreMesh(core_axis_name="c", subcore_axis_name="s"),
    compiler_params=pltpu.CompilerParams(
        use_tc_tiling_on_sc=False, needs_layout_passes=False),
    scratch_types=[
        pltpu.VMEM((SEQ, BPW), jnp.int32),
        pltpu.VMEM((NBUF, BPW, DIM), jnp.float32),
        pltpu.VMEM((NBUF, DIM // 8, 8, BPW), jnp.float32),
    ] + [pltpu.SemaphoreType.DMA] * (2 * NBUF),
)


def kernel(input_batch, table):
    out5 = _lookup(table, input_batch.T.astype(jnp.int32))
    return jnp.transpose(out5, (2, 4, 0, 1, 3)).reshape(BATCH, SEQ, DIM)
